# baseline (speedup 1.0000x reference)
import functools

import jax
import jax.numpy as jnp
from jax import lax
from jax.experimental import pallas as pl
from jax.experimental.pallas import tpu as pltpu
from jax.experimental.pallas import tpu_sc as plsc

_KEPT_FRAC = 0.001

_NC = 2
_NS = 16
_L = 16
_NW = _NC * _NS

_CHUNK = 16384
_UNROLL = 8


def _make_hist_kernel(n: int, hb: int, mode: int):
    n_per_w = n // _NW
    n_chunks = n_per_w // _CHUNK
    mesh = plsc.VectorSubcoreMesh(core_axis_name="c", subcore_axis_name="s")

    @functools.partial(
        pl.kernel,
        mesh=mesh,
        compiler_params=pltpu.CompilerParams(needs_layout_passes=False),
        out_type=jax.ShapeDtypeStruct((_NW * hb,), jnp.int32),
        scratch_types=[
            pltpu.VMEM((_CHUNK,), jnp.float32),
            pltpu.VMEM((_CHUNK,), jnp.float32),
            pltpu.VMEM((_L * hb,), jnp.int32),
            pltpu.VMEM((hb,), jnp.int32),
            pltpu.VMEM((_L,), jnp.int32),
            pltpu.SemaphoreType.DMA,
            pltpu.SemaphoreType.DMA,
        ],
    )
    def hist_kernel(pred_hbm, sel_hbm, out_hbm, buf0, buf1, hist, red, selv,
                    sem0, sem1):
        wid = lax.axis_index("s") * _NC + lax.axis_index("c")
        base = wid * n_per_w
        lane = lax.iota(jnp.int32, _L)
        ones = jnp.ones((_L,), jnp.int32)
        zeros = jnp.zeros((_L,), jnp.int32)

        def zero_body(j, c):
            hist[pl.ds(j * _L, _L)] = zeros
            return c

        lax.fori_loop(0, _L * hb // _L, zero_body, 0)

        pltpu.sync_copy(sel_hbm, selv)
        sel = selv[...]

        bufs = (buf0, buf1)
        sems = (sem0, sem1)

        def start(g, b):
            pltpu.async_copy(
                pred_hbm.at[pl.ds(base + g * _CHUNK, _CHUNK)], bufs[b], sems[b]
            )

        def wait(b):
            pltpu.make_async_copy(
                pred_hbm.at[pl.ds(0, _CHUNK)], bufs[b], sems[b]
            ).wait()

        def consume(buf):
            def vec_body(j, cc):
                for u in range(_UNROLL):
                    v = buf[pl.ds((j * _UNROLL + u) * _L, _L)]
                    bits = plsc.bitcast(v, jnp.int32)
                    if mode == 0:
                        idx = bits >> 18
                        plsc.addupdate_scatter(
                            hist, [(idx << 4) + lane], ones)
                    elif mode == 1:
                        m = (bits >> 18) == sel
                        idx = (bits >> 6) & 0xFFF
                        plsc.addupdate_scatter(
                            hist, [(idx << 4) + lane], ones, mask=m)
                    else:
                        m = (bits >> 6) == sel
                        idx = bits & 0x3F
                        plsc.addupdate_scatter(
                            hist, [(idx << 4) + lane], ones, mask=m)
                return cc

            lax.fori_loop(0, _CHUNK // (_L * _UNROLL), vec_body, 0)

        start(0, 0)

        def chunk_pair_body(g2, c):
            for b in range(2):
                g = g2 * 2 + b

                @pl.when(g + 1 < n_chunks)
                def _():
                    start(g + 1, 1 - b)

                wait(b)
                consume(bufs[b])
            return c

        lax.fori_loop(0, n_chunks // 2, chunk_pair_body, 0)

        def red_body(j, c):
            bb = (j * _L + lane) << 4
            acc = plsc.load_gather(hist, [bb + lane])
            for i in range(1, _L):
                acc = acc + plsc.load_gather(hist, [bb + ((lane + i) & (_L - 1))])
            red[pl.ds(j * _L, _L)] = acc
            return c

        lax.fori_loop(0, hb // _L, red_body, 0)
        pltpu.sync_copy(red, out_hbm.at[pl.ds(wid * hb, hb)])

    return hist_kernel


def _bce_body(t_ref, p_ref, g_ref, o_ref, acc):
    i = pl.program_id(0)
    thr = t_ref[0]
    p = p_ref[...]
    g = g_ref[...]
    log_p = jnp.maximum(jnp.log(p), -100.0)
    log_1mp = jnp.maximum(jnp.log(1.0 - p), -100.0)
    bce = -(g * log_p + (1.0 - g) * log_1mp)
    m = p < thr
    s = jnp.sum(jnp.where(m, bce, 0.0))
    c = jnp.sum(m.astype(jnp.float32))

    @pl.when(i == 0)
    def _():
        acc[0] = 0.0
        acc[1] = 0.0

    acc[0] = acc[0] + s
    acc[1] = acc[1] + c

    @pl.when(i == pl.num_programs(0) - 1)
    def _():
        o_ref[0, 0] = acc[0] / acc[1]


def _rank_step(hist, kk):
    cum = jnp.cumsum(hist)
    b = jnp.sum((cum <= kk).astype(jnp.int32)).astype(jnp.int32)
    below = jnp.where(b > 0, cum[jnp.maximum(b - 1, 0)], 0)
    return b, kk - below


def kernel(pred, target):
    n = pred.size
    k = min(int(_KEPT_FRAC * n), n - 1)
    pf = pred.reshape(-1)

    pad = (-n) % (_NW * 2 * _CHUNK)
    pf_sc = jnp.pad(pf, (0, pad), constant_values=1.0) if pad else pf
    nsc = n + pad

    hist1 = _make_hist_kernel(nsc, 4096, 0)
    hist2 = _make_hist_kernel(nsc, 4096, 1)
    hist3 = _make_hist_kernel(nsc, 64, 2)

    sel0 = jnp.zeros((_L,), jnp.int32)
    h1 = hist1(pf_sc, sel0).reshape(_NW, 4096).sum(axis=0)
    b1, k1 = _rank_step(h1, k)

    h2 = hist2(pf_sc, jnp.full((_L,), b1, jnp.int32)).reshape(_NW, 4096).sum(axis=0)
    l2, k2 = _rank_step(h2, k1)

    sel3 = (b1 << 12) | l2
    h3 = hist3(pf_sc, jnp.full((_L,), sel3, jnp.int32)).reshape(_NW, 64).sum(axis=0)
    l3, _ = _rank_step(h3, k2)

    tbits = (b1 << 18) | (l2 << 6) | l3
    thr = lax.bitcast_convert_type(tbits.astype(jnp.int32), jnp.float32)

    cols = 512
    rows = n // cols
    grid = 16
    br = rows // grid
    p2 = pf.reshape(rows, cols)
    g2 = target.reshape(rows, cols)

    grid_spec = pltpu.PrefetchScalarGridSpec(
        num_scalar_prefetch=1,
        grid=(grid,),
        in_specs=[
            pl.BlockSpec((br, cols), lambda i, s: (i, 0)),
            pl.BlockSpec((br, cols), lambda i, s: (i, 0)),
        ],
        out_specs=pl.BlockSpec(memory_space=pltpu.SMEM),
        scratch_shapes=[pltpu.SMEM((2,), jnp.float32)],
    )
    out = pl.pallas_call(
        _bce_body,
        grid_spec=grid_spec,
        out_shape=jax.ShapeDtypeStruct((1, 1), jnp.float32),
    )(thr.reshape(1), p2, g2)
    return out[0, 0]


# speedup vs baseline: 3.0599x; 3.0599x over previous
import functools

import jax
import jax.numpy as jnp
from jax import lax
from jax.experimental import pallas as pl
from jax.experimental.pallas import tpu as pltpu
from jax.experimental.pallas import tpu_sc as plsc

_KEPT_FRAC = 0.001

_NC = 2
_NS = 16
_L = 16
_NW = _NC * _NS

_CHUNK = 16384
_UNROLL = 8


def _make_hist_kernel(n: int, hb: int, mode: int):
    n_per_w = n // _NW
    n_chunks = n_per_w // _CHUNK
    mesh = plsc.VectorSubcoreMesh(core_axis_name="c", subcore_axis_name="s")

    @functools.partial(
        pl.kernel,
        mesh=mesh,
        compiler_params=pltpu.CompilerParams(needs_layout_passes=False),
        out_type=jax.ShapeDtypeStruct((_NW * hb,), jnp.int32),
        scratch_types=[
            pltpu.VMEM((_CHUNK,), jnp.float32),
            pltpu.VMEM((_CHUNK,), jnp.float32),
            pltpu.VMEM((_L * hb,), jnp.int32),
            pltpu.VMEM((hb,), jnp.int32),
            pltpu.VMEM((_L,), jnp.int32),
            pltpu.SemaphoreType.DMA,
            pltpu.SemaphoreType.DMA,
        ],
    )
    def hist_kernel(pred_hbm, sel_hbm, out_hbm, buf0, buf1, hist, red, selv,
                    sem0, sem1):
        wid = lax.axis_index("s") * _NC + lax.axis_index("c")
        base = wid * n_per_w
        lane = lax.iota(jnp.int32, _L)
        ones = jnp.ones((_L,), jnp.int32)
        zeros = jnp.zeros((_L,), jnp.int32)

        def zero_body(j, c):
            hist[pl.ds(j * _L, _L)] = zeros
            return c

        lax.fori_loop(0, _L * hb // _L, zero_body, 0)

        pltpu.sync_copy(sel_hbm, selv)
        sel = selv[...]

        bufs = (buf0, buf1)
        sems = (sem0, sem1)

        def start(g, b):
            pltpu.async_copy(
                pred_hbm.at[pl.ds(base + g * _CHUNK, _CHUNK)], bufs[b], sems[b]
            )

        def wait(b):
            pltpu.make_async_copy(
                pred_hbm.at[pl.ds(0, _CHUNK)], bufs[b], sems[b]
            ).wait()

        def consume(buf):
            @plsc.parallel_loop(0, _CHUNK // _L, unroll=_UNROLL)
            def vec_body(j):
                v = buf[pl.ds(j * _L, _L)]
                bits = plsc.bitcast(v, jnp.int32)
                if mode == 0:
                    idx = bits >> 18
                    plsc.addupdate_scatter(hist, [(idx << 4) + lane], ones)
                elif mode == 1:
                    m = (bits >> 18) == sel
                    idx = (bits >> 6) & 0xFFF
                    plsc.addupdate_scatter(
                        hist, [(idx << 4) + lane], ones, mask=m)
                else:
                    m = (bits >> 6) == sel
                    idx = bits & 0x3F
                    plsc.addupdate_scatter(
                        hist, [(idx << 4) + lane], ones, mask=m)

        start(0, 0)

        def chunk_pair_body(g2, c):
            for b in range(2):
                g = g2 * 2 + b

                @pl.when(g + 1 < n_chunks)
                def _():
                    start(g + 1, 1 - b)

                wait(b)
                consume(bufs[b])
            return c

        lax.fori_loop(0, n_chunks // 2, chunk_pair_body, 0)

        def red_body(j, c):
            bb = (j * _L + lane) << 4
            acc = plsc.load_gather(hist, [bb + lane])
            for i in range(1, _L):
                acc = acc + plsc.load_gather(hist, [bb + ((lane + i) & (_L - 1))])
            red[pl.ds(j * _L, _L)] = acc
            return c

        lax.fori_loop(0, hb // _L, red_body, 0)
        pltpu.sync_copy(red, out_hbm.at[pl.ds(wid * hb, hb)])

    return hist_kernel


def _bce_body(t_ref, p_ref, g_ref, o_ref, acc):
    i = pl.program_id(0)
    thr = t_ref[0]
    p = p_ref[...]
    g = g_ref[...]
    log_p = jnp.maximum(jnp.log(p), -100.0)
    log_1mp = jnp.maximum(jnp.log(1.0 - p), -100.0)
    bce = -(g * log_p + (1.0 - g) * log_1mp)
    m = p < thr
    s = jnp.sum(jnp.where(m, bce, 0.0))
    c = jnp.sum(m.astype(jnp.float32))

    @pl.when(i == 0)
    def _():
        acc[0] = 0.0
        acc[1] = 0.0

    acc[0] = acc[0] + s
    acc[1] = acc[1] + c

    @pl.when(i == pl.num_programs(0) - 1)
    def _():
        o_ref[0, 0] = acc[0] / acc[1]


def _rank_step(hist, kk):
    cum = jnp.cumsum(hist)
    b = jnp.sum((cum <= kk).astype(jnp.int32)).astype(jnp.int32)
    below = jnp.where(b > 0, cum[jnp.maximum(b - 1, 0)], 0)
    return b, kk - below


def kernel(pred, target):
    n = pred.size
    k = min(int(_KEPT_FRAC * n), n - 1)
    pf = pred.reshape(-1)

    pad = (-n) % (_NW * 2 * _CHUNK)
    pf_sc = jnp.pad(pf, (0, pad), constant_values=1.0) if pad else pf
    nsc = n + pad

    hist1 = _make_hist_kernel(nsc, 4096, 0)
    hist2 = _make_hist_kernel(nsc, 4096, 1)
    hist3 = _make_hist_kernel(nsc, 64, 2)

    sel0 = jnp.zeros((_L,), jnp.int32)
    h1 = hist1(pf_sc, sel0).reshape(_NW, 4096).sum(axis=0)
    b1, k1 = _rank_step(h1, k)

    h2 = hist2(pf_sc, jnp.full((_L,), b1, jnp.int32)).reshape(_NW, 4096).sum(axis=0)
    l2, k2 = _rank_step(h2, k1)

    sel3 = (b1 << 12) | l2
    h3 = hist3(pf_sc, jnp.full((_L,), sel3, jnp.int32)).reshape(_NW, 64).sum(axis=0)
    l3, _ = _rank_step(h3, k2)

    tbits = (b1 << 18) | (l2 << 6) | l3
    thr = lax.bitcast_convert_type(tbits.astype(jnp.int32), jnp.float32)

    cols = 512
    rows = n // cols
    grid = 16
    br = rows // grid
    p2 = pf.reshape(rows, cols)
    g2 = target.reshape(rows, cols)

    grid_spec = pltpu.PrefetchScalarGridSpec(
        num_scalar_prefetch=1,
        grid=(grid,),
        in_specs=[
            pl.BlockSpec((br, cols), lambda i, s: (i, 0)),
            pl.BlockSpec((br, cols), lambda i, s: (i, 0)),
        ],
        out_specs=pl.BlockSpec(memory_space=pltpu.SMEM),
        scratch_shapes=[pltpu.SMEM((2,), jnp.float32)],
    )
    out = pl.pallas_call(
        _bce_body,
        grid_spec=grid_spec,
        out_shape=jax.ShapeDtypeStruct((1, 1), jnp.float32),
    )(thr.reshape(1), p2, g2)
    return out[0, 0]


# speedup vs baseline: 3.6379x; 1.1889x over previous
import functools

import jax
import jax.numpy as jnp
from jax import lax
from jax.experimental import pallas as pl
from jax.experimental.pallas import tpu as pltpu
from jax.experimental.pallas import tpu_sc as plsc

_KEPT_FRAC = 0.001

_NC = 2
_NS = 16
_L = 16
_NW = _NC * _NS

_CHUNK = 16384
_CROWS = _CHUNK // 512
_UNROLL = 16


def _make_hist_kernel(n: int, hb: int, mode: int):
    n_per_w = n // _NW
    n_chunks = n_per_w // _CHUNK
    rows_per_w = n_per_w // 512
    mesh = plsc.VectorSubcoreMesh(core_axis_name="c", subcore_axis_name="s")

    @functools.partial(
        pl.kernel,
        mesh=mesh,
        compiler_params=pltpu.CompilerParams(
            needs_layout_passes=False, use_tc_tiling_on_sc=True),
        out_type=jax.ShapeDtypeStruct((_NW * hb,), jnp.int32),
        scratch_types=[
            pltpu.VMEM((_CROWS, 512), jnp.float32),
            pltpu.VMEM((_CROWS, 512), jnp.float32),
            pltpu.VMEM((_L * hb,), jnp.int32),
            pltpu.VMEM((hb,), jnp.int32),
            pltpu.VMEM((_L,), jnp.int32),
            pltpu.SemaphoreType.DMA,
            pltpu.SemaphoreType.DMA,
        ],
    )
    def hist_kernel(pred_hbm, sel_hbm, out_hbm, buf0, buf1, hist, red, selv,
                    sem0, sem1):
        wid = lax.axis_index("s") * _NC + lax.axis_index("c")
        base = wid * rows_per_w
        lane = lax.iota(jnp.int32, _L)
        ones = jnp.ones((_L,), jnp.int32)
        zeros = jnp.zeros((_L,), jnp.int32)

        def zero_body(j, c):
            hist[pl.ds(j * _L, _L)] = zeros
            return c

        lax.fori_loop(0, _L * hb // _L, zero_body, 0)

        pltpu.sync_copy(sel_hbm, selv)
        sel = selv[...]

        bufs = (buf0, buf1)
        sems = (sem0, sem1)

        def start(g, b):
            pltpu.async_copy(
                pred_hbm.at[pl.ds(base + g * _CROWS, _CROWS), :],
                bufs[b], sems[b]
            )

        def wait(b):
            pltpu.make_async_copy(
                pred_hbm.at[pl.ds(0, _CROWS), :], bufs[b], sems[b]
            ).wait()

        def consume(buf):
            @plsc.parallel_loop(0, _CHUNK // _L, unroll=_UNROLL)
            def vec_body(j):
                v = buf[j >> 5, pl.ds((j & 31) * _L, _L)]
                bits = plsc.bitcast(v, jnp.int32)
                if mode == 0:
                    idx = bits >> 18
                    plsc.addupdate_scatter(hist, [(idx << 4) + lane], ones)
                elif mode == 1:
                    m = (bits >> 18) == sel
                    idx = (bits >> 6) & 0xFFF
                    plsc.addupdate_scatter(
                        hist, [(idx << 4) + lane], ones, mask=m)
                else:
                    m = (bits >> 6) == sel
                    idx = bits & 0x3F
                    plsc.addupdate_scatter(
                        hist, [(idx << 4) + lane], ones, mask=m)

        start(0, 0)

        def chunk_pair_body(g2, c):
            for b in range(2):
                g = g2 * 2 + b

                @pl.when(g + 1 < n_chunks)
                def _():
                    start(g + 1, 1 - b)

                wait(b)
                consume(bufs[b])
            return c

        lax.fori_loop(0, n_chunks // 2, chunk_pair_body, 0)

        def red_body(j, c):
            bb = (j * _L + lane) << 4
            acc = plsc.load_gather(hist, [bb + lane])
            for i in range(1, _L):
                acc = acc + plsc.load_gather(hist, [bb + ((lane + i) & (_L - 1))])
            red[pl.ds(j * _L, _L)] = acc
            return c

        lax.fori_loop(0, hb // _L, red_body, 0)
        pltpu.sync_copy(red, out_hbm.at[pl.ds(wid * hb, hb)])

    return hist_kernel


def _bce_body(t_ref, p_ref, g_ref, o_ref, acc):
    i = pl.program_id(0)
    thr = t_ref[0]
    p = p_ref[...]
    g = g_ref[...]
    log_p = jnp.maximum(jnp.log(p), -100.0)
    log_1mp = jnp.maximum(jnp.log(1.0 - p), -100.0)
    bce = -(g * log_p + (1.0 - g) * log_1mp)
    m = p < thr
    s = jnp.sum(jnp.where(m, bce, 0.0))
    c = jnp.sum(m.astype(jnp.float32))

    @pl.when(i == 0)
    def _():
        acc[0] = 0.0
        acc[1] = 0.0

    acc[0] = acc[0] + s
    acc[1] = acc[1] + c

    @pl.when(i == pl.num_programs(0) - 1)
    def _():
        o_ref[0, 0] = acc[0] / acc[1]


def _rank_step(hist, kk):
    cum = jnp.cumsum(hist)
    b = jnp.sum((cum <= kk).astype(jnp.int32)).astype(jnp.int32)
    below = jnp.where(b > 0, cum[jnp.maximum(b - 1, 0)], 0)
    return b, kk - below


def kernel(pred, target):
    n = pred.size
    k = min(int(_KEPT_FRAC * n), n - 1)

    cols = 512
    rows = n // cols
    p2 = pred.reshape(rows, cols)
    g2 = target.reshape(rows, cols)

    rpad = (-rows) % (_NW * 2 * _CROWS)
    p2sc = (jnp.pad(p2, ((0, rpad), (0, 0)), constant_values=1.0)
            if rpad else p2)
    nsc = (rows + rpad) * cols

    hist1 = _make_hist_kernel(nsc, 4096, 0)
    hist2 = _make_hist_kernel(nsc, 4096, 1)
    hist3 = _make_hist_kernel(nsc, 64, 2)

    sel0 = jnp.zeros((_L,), jnp.int32)
    h1 = hist1(p2sc, sel0).reshape(_NW, 4096).sum(axis=0)
    b1, k1 = _rank_step(h1, k)

    h2 = hist2(p2sc, jnp.full((_L,), b1, jnp.int32)).reshape(_NW, 4096).sum(axis=0)
    l2, k2 = _rank_step(h2, k1)

    sel3 = (b1 << 12) | l2
    h3 = hist3(p2sc, jnp.full((_L,), sel3, jnp.int32)).reshape(_NW, 64).sum(axis=0)
    l3, _ = _rank_step(h3, k2)

    tbits = (b1 << 18) | (l2 << 6) | l3
    thr = lax.bitcast_convert_type(tbits.astype(jnp.int32), jnp.float32)

    grid = 16
    br = rows // grid

    grid_spec = pltpu.PrefetchScalarGridSpec(
        num_scalar_prefetch=1,
        grid=(grid,),
        in_specs=[
            pl.BlockSpec((br, cols), lambda i, s: (i, 0)),
            pl.BlockSpec((br, cols), lambda i, s: (i, 0)),
        ],
        out_specs=pl.BlockSpec(memory_space=pltpu.SMEM),
        scratch_shapes=[pltpu.SMEM((2,), jnp.float32)],
    )
    out = pl.pallas_call(
        _bce_body,
        grid_spec=grid_spec,
        out_shape=jax.ShapeDtypeStruct((1, 1), jnp.float32),
    )(thr.reshape(1), p2, g2)
    return out[0, 0]
